# baseline (speedup 1.0000x reference)
import functools

import jax
import jax.numpy as jnp
from jax import lax
from jax.experimental import pallas as pl
from jax.experimental.pallas import tpu as pltpu
from jax.experimental.pallas import tpu_sc as plsc

N = 10000
D = 128
E = 320000
ROWS = 10240
STRIPE = ROWS // 16
CHUNK = 128
CPW = 80
SEG = 16
NSEG = CPW // SEG
EP = 32 * CPW * CHUNK
BLK = 1024
GRID = ROWS // BLK

_mesh = plsc.VectorSubcoreMesh(
    core_axis_name="c", subcore_axis_name="s", num_cores=2, num_subcores=16)


def _deg_counts(dst4, zerosD, ones128):
    @functools.partial(
        pl.kernel,
        out_type=jax.ShapeDtypeStruct((2, ROWS, D), jnp.float32),
        mesh=_mesh,
        scratch_types=[
            pltpu.VMEM((SEG, CHUNK), jnp.int32),
            pltpu.VMEM((CHUNK, D), jnp.float32),
            pltpu.VMEM_SHARED((ROWS, D), jnp.float32),
            pltpu.SemaphoreType.DMA,
        ],
    )
    def k(dst_hbm, z_hbm, ones_hbm, out_hbm, dst_v, ones_v, acc, sa):
        c = lax.axis_index("c")
        s = lax.axis_index("s")
        w = c * 16 + s
        r0 = s * STRIPE
        pltpu.sync_copy(ones_hbm, ones_v)
        pltpu.sync_copy(z_hbm.at[pl.ds(r0, STRIPE)], acc.at[pl.ds(r0, STRIPE)])
        plsc.subcore_barrier()

        @pl.loop(0, NSEG)
        def _(g):
            pltpu.sync_copy(dst_hbm.at[w].at[pl.ds(g * SEG, SEG)], dst_v)

            for j in range(SEG):
                pltpu.async_copy(ones_v, acc.at[dst_v.at[j]], sa, add=True)
            for j in range(SEG):
                pltpu.make_async_copy(ones_v, acc.at[dst_v.at[j]], sa).wait()

        plsc.subcore_barrier()
        pltpu.sync_copy(acc.at[pl.ds(r0, STRIPE)],
                        out_hbm.at[c].at[pl.ds(r0, STRIPE)])

    return k(dst4, zerosD, ones128)


def _edge_scatter(y, src4, dst4):
    @functools.partial(
        pl.kernel,
        out_type=jax.ShapeDtypeStruct((2, ROWS, D), jnp.float32),
        mesh=_mesh,
        scratch_types=[
            pltpu.VMEM((SEG, CHUNK), jnp.int32),
            pltpu.VMEM((SEG, CHUNK), jnp.int32),
            pltpu.VMEM((CHUNK, D), jnp.float32),
            pltpu.VMEM((CHUNK, D), jnp.float32),
            pltpu.VMEM_SHARED((ROWS, D), jnp.float32),
            pltpu.SemaphoreType.DMA,
            pltpu.SemaphoreType.DMA,
            pltpu.SemaphoreType.DMA,
            pltpu.SemaphoreType.DMA,
        ],
    )
    def k(y_hbm, src_hbm, dst_hbm, out_hbm, src_v, dst_v, bufa, bufb, acc,
          ga_s, gb_s, sa_s, sb_s):
        c = lax.axis_index("c")
        s = lax.axis_index("s")
        w = c * 16 + s
        r0 = s * STRIPE
        pltpu.sync_copy(y_hbm.at[pl.ds(r0, STRIPE)], acc.at[pl.ds(r0, STRIPE)])
        plsc.subcore_barrier()

        @pl.loop(0, NSEG)
        def _(g):
            pltpu.sync_copy(src_hbm.at[w].at[pl.ds(g * SEG, SEG)], src_v)
            pltpu.sync_copy(dst_hbm.at[w].at[pl.ds(g * SEG, SEG)], dst_v)

            @pl.loop(0, SEG // 2)
            def _(p):
                ga = pltpu.async_copy(y_hbm.at[src_v.at[2 * p]], bufa, ga_s)
                gb = pltpu.async_copy(
                    y_hbm.at[src_v.at[2 * p + 1]], bufb, gb_s)
                ga.wait()
                sa = pltpu.async_copy(
                    bufa, acc.at[dst_v.at[2 * p]], sa_s, add=True)
                gb.wait()
                sb = pltpu.async_copy(
                    bufb, acc.at[dst_v.at[2 * p + 1]], sb_s, add=True)
                sa.wait()
                sb.wait()

        plsc.subcore_barrier()
        pltpu.sync_copy(acc.at[pl.ds(r0, STRIPE)],
                        out_hbm.at[c].at[pl.ds(r0, STRIPE)])

    return k(y, src4, dst4)


_row_spec = pl.BlockSpec((BLK, D), lambda i: (i, 0))
_pair_spec = pl.BlockSpec((2, BLK, D), lambda i: (0, i, 0))
_w_spec = pl.BlockSpec((D, D), lambda i: (0, 0))
_vec_spec = pl.BlockSpec((1, D), lambda i: (0, 0))
_deg_spec = pl.BlockSpec((2, BLK, D), lambda i: (0, i, 0))


def _layer_norm(h, g, b):
    mu = jnp.mean(h, axis=1, keepdims=True)
    var = jnp.mean((h - mu) ** 2, axis=1, keepdims=True)
    return (h - mu) / jnp.sqrt(var + 1e-5) * g + b


def _tc_matmul0(xp, W0):
    def body(x_ref, w_ref, o_ref):
        o_ref[...] = jnp.dot(
            x_ref[...], w_ref[...], preferred_element_type=jnp.float32)

    return pl.pallas_call(
        body,
        grid=(GRID,),
        in_specs=[_row_spec, _w_spec],
        out_specs=_row_spec,
        out_shape=jax.ShapeDtypeStruct((ROWS, D), jnp.float32),
    )(xp, W0)


def _tc_first(xw0, degp):
    def body(xw_ref, dg_ref, dis_ref, y_ref):
        deg = dg_ref[0, :, 0:1] + dg_ref[1, :, 0:1] + 1.0
        dis = lax.rsqrt(deg)
        dis_ref[...] = jnp.broadcast_to(dis, (BLK, D))
        y_ref[...] = xw_ref[...] * dis

    return pl.pallas_call(
        body,
        grid=(GRID,),
        in_specs=[_row_spec, _deg_spec],
        out_specs=[_row_spec, _row_spec],
        out_shape=[
            jax.ShapeDtypeStruct((ROWS, D), jnp.float32),
            jax.ShapeDtypeStruct((ROWS, D), jnp.float32),
        ],
    )(xw0, degp)


def _tc_mid1(s0, y0, dis_b, b0, g1, be1, W1):
    def body(s_ref, y0_ref, dis_ref, b_ref, g_ref, be_ref, w_ref,
             h1_ref, y_ref):
        dis = dis_ref[...]
        agg = s_ref[0] + s_ref[1] - y0_ref[...]
        h1 = jnp.maximum(dis * agg + b_ref[...], 0.0)
        h1_ref[...] = h1
        t = _layer_norm(h1, g_ref[...], be_ref[...])
        y_ref[...] = jnp.dot(
            t, w_ref[...], preferred_element_type=jnp.float32) * dis

    return pl.pallas_call(
        body,
        grid=(GRID,),
        in_specs=[_pair_spec, _row_spec, _row_spec, _vec_spec, _vec_spec,
                  _vec_spec, _w_spec],
        out_specs=[_row_spec, _row_spec],
        out_shape=[
            jax.ShapeDtypeStruct((ROWS, D), jnp.float32),
            jax.ShapeDtypeStruct((ROWS, D), jnp.float32),
        ],
    )(s0, y0, dis_b, b0, g1, be1, W1)


def _tc_mid2(s1, y1, dis_b, b1, h1, W2):
    def body(s_ref, y1_ref, dis_ref, b_ref, h1_ref, w_ref, y_ref):
        dis = dis_ref[...]
        agg = s_ref[0] + s_ref[1] - y1_ref[...]
        h2 = jnp.maximum(dis * agg + b_ref[...], 0.0) + h1_ref[...]
        y_ref[...] = jnp.dot(
            h2, w_ref[...], preferred_element_type=jnp.float32) * dis

    return pl.pallas_call(
        body,
        grid=(GRID,),
        in_specs=[_pair_spec, _row_spec, _row_spec, _vec_spec, _row_spec,
                  _w_spec],
        out_specs=_row_spec,
        out_shape=jax.ShapeDtypeStruct((ROWS, D), jnp.float32),
    )(s1, y1, dis_b, b1, h1, W2)


def _tc_final(s2, y2, dis_b, b2, gf, bef):
    def body(s_ref, y2_ref, dis_ref, b_ref, g_ref, be_ref, o_ref):
        agg = s_ref[0] + s_ref[1] - y2_ref[...]
        h3 = dis_ref[...] * agg + b_ref[...]
        o_ref[...] = _layer_norm(h3, g_ref[...], be_ref[...])

    return pl.pallas_call(
        body,
        grid=(GRID,),
        in_specs=[_pair_spec, _row_spec, _row_spec, _vec_spec, _vec_spec,
                  _vec_spec],
        out_specs=_row_spec,
        out_shape=jax.ShapeDtypeStruct((ROWS, D), jnp.float32),
    )(s2, y2, dis_b, b2, gf, bef)


def kernel(x, edge_index, W0, b0, W1, b1, W2, b2, ln1_g, ln1_b, lnf_g, lnf_b):
    ei = edge_index.astype(jnp.int32)
    src, dst = ei[0], ei[1]
    padn = EP - E
    pad_i = jnp.arange(padn, dtype=jnp.int32)
    pad_src = (pad_i * 97) % N
    pad_dst = N + pad_i % (ROWS - N)
    src4 = jnp.concatenate([src, pad_src]).reshape(32, CPW, CHUNK)
    dst4 = jnp.concatenate([dst, pad_dst]).reshape(32, CPW, CHUNK)
    xp = jnp.pad(x, ((0, ROWS - N), (0, 0)))
    zerosD = jnp.zeros((ROWS, D), jnp.float32)
    ones128 = jnp.ones((CHUNK, D), jnp.float32)
    b0r = b0.reshape(1, D)
    b1r = b1.reshape(1, D)
    b2r = b2.reshape(1, D)
    g1r = ln1_g.reshape(1, D)
    be1r = ln1_b.reshape(1, D)
    gfr = lnf_g.reshape(1, D)
    befr = lnf_b.reshape(1, D)

    degp = _deg_counts(dst4, zerosD, ones128)
    xw0 = _tc_matmul0(xp, W0)
    dis_b, y0 = _tc_first(xw0, degp)
    s0 = _edge_scatter(y0, src4, dst4)
    h1, y1 = _tc_mid1(s0, y0, dis_b, b0r, g1r, be1r, W1)
    s1 = _edge_scatter(y1, src4, dst4)
    y2 = _tc_mid2(s1, y1, dis_b, b1r, h1, W2)
    s2 = _edge_scatter(y2, src4, dst4)
    out = _tc_final(s2, y2, dis_b, b2r, gfr, befr)
    return out[:N]


# speedup vs baseline: 1.0280x; 1.0280x over previous
import functools

import jax
import jax.numpy as jnp
from jax import lax
from jax.experimental import pallas as pl
from jax.experimental.pallas import tpu as pltpu
from jax.experimental.pallas import tpu_sc as plsc

N = 10000
D = 128
E = 320000
ROWS = 10240
STRIPE = ROWS // 16
CHUNK = 128
CPW = 80
SEG = 40
NSEG = CPW // SEG
EP = 32 * CPW * CHUNK
BLK = 1024
GRID = ROWS // BLK

_mesh = plsc.VectorSubcoreMesh(
    core_axis_name="c", subcore_axis_name="s", num_cores=2, num_subcores=16)


def _deg_counts(dst4, zerosD, ones128):
    @functools.partial(
        pl.kernel,
        out_type=jax.ShapeDtypeStruct((2, ROWS, D), jnp.float32),
        mesh=_mesh,
        scratch_types=[
            pltpu.VMEM((SEG, CHUNK), jnp.int32),
            pltpu.VMEM((CHUNK, D), jnp.float32),
            pltpu.VMEM_SHARED((ROWS, D), jnp.float32),
            pltpu.SemaphoreType.DMA,
        ],
    )
    def k(dst_hbm, z_hbm, ones_hbm, out_hbm, dst_v, ones_v, acc, sa):
        c = lax.axis_index("c")
        s = lax.axis_index("s")
        w = c * 16 + s
        r0 = s * STRIPE
        pltpu.sync_copy(ones_hbm, ones_v)
        pltpu.sync_copy(z_hbm, acc.at[pl.ds(r0, STRIPE)])
        plsc.subcore_barrier()

        @pl.loop(0, NSEG)
        def _(g):
            pltpu.sync_copy(dst_hbm.at[w].at[pl.ds(g * SEG, SEG)], dst_v)

            @pl.loop(0, SEG // 8)
            def _(q):
                for i in range(8):
                    pltpu.async_copy(
                        ones_v, acc.at[dst_v.at[q * 8 + i]], sa, add=True)
                for i in range(8):
                    pltpu.make_async_copy(
                        ones_v, acc.at[dst_v.at[q * 8 + i]], sa).wait()

        plsc.subcore_barrier()
        pltpu.sync_copy(acc.at[pl.ds(r0, STRIPE)],
                        out_hbm.at[c].at[pl.ds(r0, STRIPE)])

    return k(dst4, zerosD, ones128)


def _edge_scatter(y, src4, dst4):
    @functools.partial(
        pl.kernel,
        out_type=jax.ShapeDtypeStruct((2, ROWS, D), jnp.float32),
        mesh=_mesh,
        scratch_types=[
            pltpu.VMEM((SEG, CHUNK), jnp.int32),
            pltpu.VMEM((SEG, CHUNK), jnp.int32),
            pltpu.VMEM((CHUNK, D), jnp.float32),
            pltpu.VMEM((CHUNK, D), jnp.float32),
            pltpu.VMEM_SHARED((ROWS, D), jnp.float32),
            pltpu.SemaphoreType.DMA,
            pltpu.SemaphoreType.DMA,
            pltpu.SemaphoreType.DMA,
            pltpu.SemaphoreType.DMA,
        ],
    )
    def k(y_hbm, src_hbm, dst_hbm, out_hbm, src_v, dst_v, bufa, bufb, acc,
          ga_s, gb_s, sa_s, sb_s):
        c = lax.axis_index("c")
        s = lax.axis_index("s")
        w = c * 16 + s
        r0 = s * STRIPE
        pltpu.sync_copy(y_hbm.at[pl.ds(r0, STRIPE)], acc.at[pl.ds(r0, STRIPE)])
        plsc.subcore_barrier()

        @pl.loop(0, NSEG)
        def _(g):
            pltpu.sync_copy(src_hbm.at[w].at[pl.ds(g * SEG, SEG)], src_v)
            pltpu.sync_copy(dst_hbm.at[w].at[pl.ds(g * SEG, SEG)], dst_v)

            @pl.loop(0, SEG // 2)
            def _(p):
                ga = pltpu.async_copy(y_hbm.at[src_v.at[2 * p]], bufa, ga_s)
                gb = pltpu.async_copy(
                    y_hbm.at[src_v.at[2 * p + 1]], bufb, gb_s)
                ga.wait()
                sa = pltpu.async_copy(
                    bufa, acc.at[dst_v.at[2 * p]], sa_s, add=True)
                gb.wait()
                sb = pltpu.async_copy(
                    bufb, acc.at[dst_v.at[2 * p + 1]], sb_s, add=True)
                sa.wait()
                sb.wait()

        plsc.subcore_barrier()
        pltpu.sync_copy(acc.at[pl.ds(r0, STRIPE)],
                        out_hbm.at[c].at[pl.ds(r0, STRIPE)])

    return k(y, src4, dst4)


_row_spec = pl.BlockSpec((BLK, D), lambda i: (i, 0))
_pair_spec = pl.BlockSpec((2, BLK, D), lambda i: (0, i, 0))
_w_spec = pl.BlockSpec((D, D), lambda i: (0, 0))
_vec_spec = pl.BlockSpec((1, D), lambda i: (0, 0))
_deg_spec = pl.BlockSpec((2, BLK, D), lambda i: (0, i, 0))


def _layer_norm(h, g, b):
    mu = jnp.mean(h, axis=1, keepdims=True)
    var = jnp.mean((h - mu) ** 2, axis=1, keepdims=True)
    return (h - mu) / jnp.sqrt(var + 1e-5) * g + b


def _tc_matmul0(xp, W0):
    def body(x_ref, w_ref, o_ref):
        o_ref[...] = jnp.dot(
            x_ref[...], w_ref[...], preferred_element_type=jnp.float32)

    return pl.pallas_call(
        body,
        grid=(GRID,),
        in_specs=[_row_spec, _w_spec],
        out_specs=_row_spec,
        out_shape=jax.ShapeDtypeStruct((ROWS, D), jnp.float32),
    )(xp, W0)


def _tc_first(xw0, degp):
    def body(xw_ref, dg_ref, dis_ref, y_ref):
        deg = dg_ref[0, :, 0:1] + dg_ref[1, :, 0:1] + 1.0
        dis = lax.rsqrt(deg)
        dis_ref[...] = jnp.broadcast_to(dis, (BLK, D))
        y_ref[...] = xw_ref[...] * dis

    return pl.pallas_call(
        body,
        grid=(GRID,),
        in_specs=[_row_spec, _deg_spec],
        out_specs=[_row_spec, _row_spec],
        out_shape=[
            jax.ShapeDtypeStruct((ROWS, D), jnp.float32),
            jax.ShapeDtypeStruct((ROWS, D), jnp.float32),
        ],
    )(xw0, degp)


def _tc_mid1(s0, y0, dis_b, b0, g1, be1, W1):
    def body(s_ref, y0_ref, dis_ref, b_ref, g_ref, be_ref, w_ref,
             h1_ref, y_ref):
        dis = dis_ref[...]
        agg = s_ref[0] + s_ref[1] - y0_ref[...]
        h1 = jnp.maximum(dis * agg + b_ref[...], 0.0)
        h1_ref[...] = h1
        t = _layer_norm(h1, g_ref[...], be_ref[...])
        y_ref[...] = jnp.dot(
            t, w_ref[...], preferred_element_type=jnp.float32) * dis

    return pl.pallas_call(
        body,
        grid=(GRID,),
        in_specs=[_pair_spec, _row_spec, _row_spec, _vec_spec, _vec_spec,
                  _vec_spec, _w_spec],
        out_specs=[_row_spec, _row_spec],
        out_shape=[
            jax.ShapeDtypeStruct((ROWS, D), jnp.float32),
            jax.ShapeDtypeStruct((ROWS, D), jnp.float32),
        ],
    )(s0, y0, dis_b, b0, g1, be1, W1)


def _tc_mid2(s1, y1, dis_b, b1, h1, W2):
    def body(s_ref, y1_ref, dis_ref, b_ref, h1_ref, w_ref, y_ref):
        dis = dis_ref[...]
        agg = s_ref[0] + s_ref[1] - y1_ref[...]
        h2 = jnp.maximum(dis * agg + b_ref[...], 0.0) + h1_ref[...]
        y_ref[...] = jnp.dot(
            h2, w_ref[...], preferred_element_type=jnp.float32) * dis

    return pl.pallas_call(
        body,
        grid=(GRID,),
        in_specs=[_pair_spec, _row_spec, _row_spec, _vec_spec, _row_spec,
                  _w_spec],
        out_specs=_row_spec,
        out_shape=jax.ShapeDtypeStruct((ROWS, D), jnp.float32),
    )(s1, y1, dis_b, b1, h1, W2)


def _tc_final(s2, y2, dis_b, b2, gf, bef):
    def body(s_ref, y2_ref, dis_ref, b_ref, g_ref, be_ref, o_ref):
        agg = s_ref[0] + s_ref[1] - y2_ref[...]
        h3 = dis_ref[...] * agg + b_ref[...]
        o_ref[...] = _layer_norm(h3, g_ref[...], be_ref[...])

    return pl.pallas_call(
        body,
        grid=(GRID,),
        in_specs=[_pair_spec, _row_spec, _row_spec, _vec_spec, _vec_spec,
                  _vec_spec],
        out_specs=_row_spec,
        out_shape=jax.ShapeDtypeStruct((ROWS, D), jnp.float32),
    )(s2, y2, dis_b, b2, gf, bef)


def kernel(x, edge_index, W0, b0, W1, b1, W2, b2, ln1_g, ln1_b, lnf_g, lnf_b):
    ei = edge_index.astype(jnp.int32)
    src, dst = ei[0], ei[1]
    padn = EP - E
    pad_i = jnp.arange(padn, dtype=jnp.int32)
    pad_src = (pad_i * 97) % N
    pad_dst = N + pad_i % (ROWS - N)
    src4 = jnp.concatenate([src, pad_src]).reshape(32, CPW, CHUNK)
    dst4 = jnp.concatenate([dst, pad_dst]).reshape(32, CPW, CHUNK)
    xp = jnp.pad(x, ((0, ROWS - N), (0, 0)))
    zerosD = jnp.zeros((STRIPE, D), jnp.float32)
    ones128 = jnp.ones((CHUNK, D), jnp.float32)
    b0r = b0.reshape(1, D)
    b1r = b1.reshape(1, D)
    b2r = b2.reshape(1, D)
    g1r = ln1_g.reshape(1, D)
    be1r = ln1_b.reshape(1, D)
    gfr = lnf_g.reshape(1, D)
    befr = lnf_b.reshape(1, D)

    degp = _deg_counts(dst4, zerosD, ones128)
    xw0 = _tc_matmul0(xp, W0)
    dis_b, y0 = _tc_first(xw0, degp)
    s0 = _edge_scatter(y0, src4, dst4)
    h1, y1 = _tc_mid1(s0, y0, dis_b, b0r, g1r, be1r, W1)
    s1 = _edge_scatter(y1, src4, dst4)
    y2 = _tc_mid2(s1, y1, dis_b, b1r, h1, W2)
    s2 = _edge_scatter(y2, src4, dst4)
    out = _tc_final(s2, y2, dis_b, b2r, gfr, befr)
    return out[:N]


# speedup vs baseline: 1.0409x; 1.0126x over previous
import functools

import jax
import jax.numpy as jnp
from jax import lax
from jax.experimental import pallas as pl
from jax.experimental.pallas import tpu as pltpu
from jax.experimental.pallas import tpu_sc as plsc

N = 10000
D = 128
E = 320000
ROWS = 10240
STRIPE = ROWS // 16
CHUNK = 128
CPW = 80
SEG = 40
NSEG = CPW // SEG
EP = 32 * CPW * CHUNK
BLK = 1024
GRID = ROWS // BLK

_mesh = plsc.VectorSubcoreMesh(
    core_axis_name="c", subcore_axis_name="s", num_cores=2, num_subcores=16)


def _deg_counts(dst4, zerosD, ones128):
    @functools.partial(
        pl.kernel,
        out_type=jax.ShapeDtypeStruct((2, ROWS, D), jnp.float32),
        mesh=_mesh,
        scratch_types=[
            pltpu.VMEM((SEG, CHUNK), jnp.int32),
            pltpu.VMEM((CHUNK, D), jnp.float32),
            pltpu.VMEM_SHARED((ROWS, D), jnp.float32),
            pltpu.SemaphoreType.DMA,
            pltpu.SemaphoreType.DMA,
        ],
    )
    def k(dst_hbm, z_hbm, ones_hbm, out_hbm, dst_v, ones_v, acc, sa, i_s):
        c = lax.axis_index("c")
        s = lax.axis_index("s")
        w = c * 16 + s
        r0 = s * STRIPE
        ini = pltpu.async_copy(z_hbm, acc.at[pl.ds(r0, STRIPE)], i_s)
        pltpu.sync_copy(ones_hbm, ones_v)
        pltpu.sync_copy(dst_hbm.at[w].at[pl.ds(0, SEG)], dst_v)
        ini.wait()
        plsc.subcore_barrier()

        @pl.loop(0, NSEG)
        def _(g):
            @pl.when(g > 0)
            def _():
                pltpu.sync_copy(dst_hbm.at[w].at[pl.ds(g * SEG, SEG)], dst_v)

            @pl.loop(0, SEG // 8)
            def _(q):
                for i in range(8):
                    pltpu.async_copy(
                        ones_v, acc.at[dst_v.at[q * 8 + i]], sa, add=True)
                for i in range(8):
                    pltpu.make_async_copy(
                        ones_v, acc.at[dst_v.at[q * 8 + i]], sa).wait()

        plsc.subcore_barrier()
        pltpu.sync_copy(acc.at[pl.ds(r0, STRIPE)],
                        out_hbm.at[c].at[pl.ds(r0, STRIPE)])

    return k(dst4, zerosD, ones128)


def _edge_scatter(y, src4, dst4):
    @functools.partial(
        pl.kernel,
        out_type=jax.ShapeDtypeStruct((2, ROWS, D), jnp.float32),
        mesh=_mesh,
        scratch_types=[
            pltpu.VMEM((SEG, CHUNK), jnp.int32),
            pltpu.VMEM((SEG, CHUNK), jnp.int32),
            pltpu.VMEM((CHUNK, D), jnp.float32),
            pltpu.VMEM((CHUNK, D), jnp.float32),
            pltpu.VMEM_SHARED((ROWS, D), jnp.float32),
            pltpu.SemaphoreType.DMA,
            pltpu.SemaphoreType.DMA,
            pltpu.SemaphoreType.DMA,
            pltpu.SemaphoreType.DMA,
            pltpu.SemaphoreType.DMA,
        ],
    )
    def k(y_hbm, src_hbm, dst_hbm, out_hbm, src_v, dst_v, bufa, bufb, acc,
          ga_s, gb_s, sa_s, sb_s, i_s):
        c = lax.axis_index("c")
        s = lax.axis_index("s")
        w = c * 16 + s
        r0 = s * STRIPE
        ini = pltpu.async_copy(
            y_hbm.at[pl.ds(r0, STRIPE)], acc.at[pl.ds(r0, STRIPE)], i_s)
        pltpu.sync_copy(src_hbm.at[w].at[pl.ds(0, SEG)], src_v)
        pltpu.sync_copy(dst_hbm.at[w].at[pl.ds(0, SEG)], dst_v)
        ini.wait()
        plsc.subcore_barrier()

        @pl.loop(0, NSEG)
        def _(g):
            @pl.when(g > 0)
            def _():
                pltpu.sync_copy(src_hbm.at[w].at[pl.ds(g * SEG, SEG)], src_v)
                pltpu.sync_copy(dst_hbm.at[w].at[pl.ds(g * SEG, SEG)], dst_v)

            @pl.loop(0, SEG // 2)
            def _(p):
                ga = pltpu.async_copy(y_hbm.at[src_v.at[2 * p]], bufa, ga_s)
                gb = pltpu.async_copy(
                    y_hbm.at[src_v.at[2 * p + 1]], bufb, gb_s)
                ga.wait()
                sa = pltpu.async_copy(
                    bufa, acc.at[dst_v.at[2 * p]], sa_s, add=True)
                gb.wait()
                sb = pltpu.async_copy(
                    bufb, acc.at[dst_v.at[2 * p + 1]], sb_s, add=True)
                sa.wait()
                sb.wait()

        plsc.subcore_barrier()
        pltpu.sync_copy(acc.at[pl.ds(r0, STRIPE)],
                        out_hbm.at[c].at[pl.ds(r0, STRIPE)])

    return k(y, src4, dst4)


_row_spec = pl.BlockSpec((BLK, D), lambda i: (i, 0))
_pair_spec = pl.BlockSpec((2, BLK, D), lambda i: (0, i, 0))
_w_spec = pl.BlockSpec((D, D), lambda i: (0, 0))
_vec_spec = pl.BlockSpec((1, D), lambda i: (0, 0))
_deg_spec = pl.BlockSpec((2, BLK, D), lambda i: (0, i, 0))


def _layer_norm(h, g, b):
    mu = jnp.mean(h, axis=1, keepdims=True)
    var = jnp.mean((h - mu) ** 2, axis=1, keepdims=True)
    return (h - mu) / jnp.sqrt(var + 1e-5) * g + b


def _tc_matmul0(xp, W0):
    def body(x_ref, w_ref, o_ref):
        o_ref[...] = jnp.dot(
            x_ref[...], w_ref[...], preferred_element_type=jnp.float32)

    return pl.pallas_call(
        body,
        grid=(GRID,),
        in_specs=[_row_spec, _w_spec],
        out_specs=_row_spec,
        out_shape=jax.ShapeDtypeStruct((ROWS, D), jnp.float32),
    )(xp, W0)


def _tc_first(xw0, degp):
    def body(xw_ref, dg_ref, dis_ref, y_ref):
        deg = dg_ref[0, :, 0:1] + dg_ref[1, :, 0:1] + 1.0
        dis = lax.rsqrt(deg)
        dis_ref[...] = jnp.broadcast_to(dis, (BLK, D))
        y_ref[...] = xw_ref[...] * dis

    return pl.pallas_call(
        body,
        grid=(GRID,),
        in_specs=[_row_spec, _deg_spec],
        out_specs=[_row_spec, _row_spec],
        out_shape=[
            jax.ShapeDtypeStruct((ROWS, D), jnp.float32),
            jax.ShapeDtypeStruct((ROWS, D), jnp.float32),
        ],
    )(xw0, degp)


def _tc_mid1(s0, y0, dis_b, b0, g1, be1, W1):
    def body(s_ref, y0_ref, dis_ref, b_ref, g_ref, be_ref, w_ref,
             h1_ref, y_ref):
        dis = dis_ref[...]
        agg = s_ref[0] + s_ref[1] - y0_ref[...]
        h1 = jnp.maximum(dis * agg + b_ref[...], 0.0)
        h1_ref[...] = h1
        t = _layer_norm(h1, g_ref[...], be_ref[...])
        y_ref[...] = jnp.dot(
            t, w_ref[...], preferred_element_type=jnp.float32) * dis

    return pl.pallas_call(
        body,
        grid=(GRID,),
        in_specs=[_pair_spec, _row_spec, _row_spec, _vec_spec, _vec_spec,
                  _vec_spec, _w_spec],
        out_specs=[_row_spec, _row_spec],
        out_shape=[
            jax.ShapeDtypeStruct((ROWS, D), jnp.float32),
            jax.ShapeDtypeStruct((ROWS, D), jnp.float32),
        ],
    )(s0, y0, dis_b, b0, g1, be1, W1)


def _tc_mid2(s1, y1, dis_b, b1, h1, W2):
    def body(s_ref, y1_ref, dis_ref, b_ref, h1_ref, w_ref, y_ref):
        dis = dis_ref[...]
        agg = s_ref[0] + s_ref[1] - y1_ref[...]
        h2 = jnp.maximum(dis * agg + b_ref[...], 0.0) + h1_ref[...]
        y_ref[...] = jnp.dot(
            h2, w_ref[...], preferred_element_type=jnp.float32) * dis

    return pl.pallas_call(
        body,
        grid=(GRID,),
        in_specs=[_pair_spec, _row_spec, _row_spec, _vec_spec, _row_spec,
                  _w_spec],
        out_specs=_row_spec,
        out_shape=jax.ShapeDtypeStruct((ROWS, D), jnp.float32),
    )(s1, y1, dis_b, b1, h1, W2)


def _tc_final(s2, y2, dis_b, b2, gf, bef):
    def body(s_ref, y2_ref, dis_ref, b_ref, g_ref, be_ref, o_ref):
        agg = s_ref[0] + s_ref[1] - y2_ref[...]
        h3 = dis_ref[...] * agg + b_ref[...]
        o_ref[...] = _layer_norm(h3, g_ref[...], be_ref[...])

    return pl.pallas_call(
        body,
        grid=(GRID,),
        in_specs=[_pair_spec, _row_spec, _row_spec, _vec_spec, _vec_spec,
                  _vec_spec],
        out_specs=_row_spec,
        out_shape=jax.ShapeDtypeStruct((ROWS, D), jnp.float32),
    )(s2, y2, dis_b, b2, gf, bef)


def kernel(x, edge_index, W0, b0, W1, b1, W2, b2, ln1_g, ln1_b, lnf_g, lnf_b):
    ei = edge_index.astype(jnp.int32)
    src, dst = ei[0], ei[1]
    padn = EP - E
    pad_i = jnp.arange(padn, dtype=jnp.int32)
    pad_src = (pad_i * 97) % N
    pad_dst = N + pad_i % (ROWS - N)
    src4 = jnp.concatenate([src, pad_src]).reshape(32, CPW, CHUNK)
    dst4 = jnp.concatenate([dst, pad_dst]).reshape(32, CPW, CHUNK)
    xp = jnp.pad(x, ((0, ROWS - N), (0, 0)))
    zerosD = jnp.zeros((STRIPE, D), jnp.float32)
    ones128 = jnp.ones((CHUNK, D), jnp.float32)
    b0r = b0.reshape(1, D)
    b1r = b1.reshape(1, D)
    b2r = b2.reshape(1, D)
    g1r = ln1_g.reshape(1, D)
    be1r = ln1_b.reshape(1, D)
    gfr = lnf_g.reshape(1, D)
    befr = lnf_b.reshape(1, D)

    degp = _deg_counts(dst4, zerosD, ones128)
    xw0 = _tc_matmul0(xp, W0)
    dis_b, y0 = _tc_first(xw0, degp)
    s0 = _edge_scatter(y0, src4, dst4)
    h1, y1 = _tc_mid1(s0, y0, dis_b, b0r, g1r, be1r, W1)
    s1 = _edge_scatter(y1, src4, dst4)
    y2 = _tc_mid2(s1, y1, dis_b, b1r, h1, W2)
    s2 = _edge_scatter(y2, src4, dst4)
    out = _tc_final(s2, y2, dis_b, b2r, gfr, befr)
    return out[:N]
